# Initial kernel scaffold; baseline (speedup 1.0000x reference)
import jax
import jax.numpy as jnp
from jax.experimental import pallas as pl


def kernel(x, edge_index, conv11_w, conv11_b, conv21_w, conv21_b, conv31_w, conv31_b, fc11_w, fc11_b, fc12_w, fc12_b, fc13_w, fc13_b, fc21_w, fc21_b, fc22_w, fc22_b, fc23_w, fc23_b, fc31_w, fc31_b, fc32_w, fc32_b, fc33_w, fc33_b, fc2_w, fc2_b):
    raise NotImplementedError("write your pallas kernel here")


# speedup vs baseline: 9.5749x; 9.5749x over previous
import jax
import jax.numpy as jnp
from jax import lax
from jax.experimental import pallas as pl
from jax.experimental.pallas import tpu as pltpu
from jax.experimental.pallas import tpu_sc as plsc

_CHUNK = 128
_F = 32


def _relu(v):
    return jnp.maximum(v, 0.0)


def _seg_sum_sc(y, src3, dst3, zeros, n_pad, n_chunks, nc, ns):
    rpt = n_pad // ns

    mesh = plsc.VectorSubcoreMesh(core_axis_name="c", subcore_axis_name="s")

    def body(y_hbm, src_hbm, dst_hbm, z_hbm, out_hbm,
             src_v, dst_v, rows_v, acc_sh, sem):
        c = lax.axis_index("c")
        s = lax.axis_index("s")
        wid = c * ns + s
        pltpu.sync_copy(z_hbm.at[pl.ds(s * rpt, rpt)],
                        acc_sh.at[pl.ds(s * rpt, rpt)])
        pltpu.sync_copy(src_hbm.at[wid], src_v)
        pltpu.sync_copy(dst_hbm.at[wid], dst_v)
        plsc.subcore_barrier()

        def step(j, carry):
            pltpu.async_copy(y_hbm.at[src_v.at[j]], rows_v, sem).wait()
            pltpu.sync_copy(rows_v, acc_sh.at[dst_v.at[j]], add=True)
            return carry

        lax.fori_loop(0, n_chunks, step, 0)
        plsc.subcore_barrier()
        pltpu.sync_copy(acc_sh.at[pl.ds(s * rpt, rpt)],
                        out_hbm.at[c, pl.ds(s * rpt, rpt)])

    f = pl.kernel(
        body,
        out_type=jax.ShapeDtypeStruct((nc, n_pad, _F), jnp.float32),
        mesh=mesh,
        scratch_types=[
            pltpu.VMEM((n_chunks, _CHUNK), jnp.int32),
            pltpu.VMEM((n_chunks, _CHUNK), jnp.int32),
            pltpu.VMEM((_CHUNK, _F), jnp.float32),
            pltpu.VMEM_SHARED((n_pad, _F), jnp.float32),
            pltpu.SemaphoreType.DMA,
        ],
        compiler_params=pltpu.CompilerParams(use_tc_tiling_on_sc=False),
    )
    return f(y, src3, dst3, zeros)


def _tc_pre(x, wcat, bcat, rblk):
    n_pad, din = x.shape

    def body(x_ref, w_ref, b_ref, y_ref, s_ref):
        h = jnp.dot(x_ref[...], w_ref[...], precision=lax.Precision.HIGHEST,
                    preferred_element_type=jnp.float32) + b_ref[...]
        y_ref[...] = h[:, 3 * _F:4 * _F]
        s_ref[...] = _relu(h[:, 0:_F]) + _relu(h[:, _F:2 * _F] * h[:, 2 * _F:3 * _F])

    return pl.pallas_call(
        body,
        grid=(n_pad // rblk,),
        in_specs=[pl.BlockSpec((rblk, din), lambda i: (i, 0)),
                  pl.BlockSpec((din, 4 * _F), lambda i: (0, 0)),
                  pl.BlockSpec((1, 4 * _F), lambda i: (0, 0))],
        out_specs=[pl.BlockSpec((rblk, _F), lambda i: (i, 0)),
                   pl.BlockSpec((rblk, _F), lambda i: (i, 0))],
        out_shape=[jax.ShapeDtypeStruct((n_pad, _F), jnp.float32),
                   jax.ShapeDtypeStruct((n_pad, _F), jnp.float32)],
    )(x, wcat, bcat)


def _tc_mid(s_prev, p, cb, wcat, bcat, rblk):
    n_pad = s_prev.shape[0]

    def body(s_ref, p_ref, cb_ref, w_ref, b_ref, y_ref, s2_ref):
        xk = s_ref[...] + _relu(p_ref[0] + p_ref[1] + cb_ref[...])
        h = jnp.dot(xk, w_ref[...], precision=lax.Precision.HIGHEST,
                    preferred_element_type=jnp.float32) + b_ref[...]
        y_ref[...] = h[:, 3 * _F:4 * _F]
        s2_ref[...] = _relu(h[:, 0:_F]) + _relu(h[:, _F:2 * _F] * h[:, 2 * _F:3 * _F])

    return pl.pallas_call(
        body,
        grid=(n_pad // rblk,),
        in_specs=[pl.BlockSpec((rblk, _F), lambda i: (i, 0)),
                  pl.BlockSpec((2, rblk, _F), lambda i: (0, i, 0)),
                  pl.BlockSpec((1, _F), lambda i: (0, 0)),
                  pl.BlockSpec((_F, 4 * _F), lambda i: (0, 0)),
                  pl.BlockSpec((1, 4 * _F), lambda i: (0, 0))],
        out_specs=[pl.BlockSpec((rblk, _F), lambda i: (i, 0)),
                   pl.BlockSpec((rblk, _F), lambda i: (i, 0))],
        out_shape=[jax.ShapeDtypeStruct((n_pad, _F), jnp.float32),
                   jax.ShapeDtypeStruct((n_pad, _F), jnp.float32)],
    )(s_prev, p, cb, wcat, bcat)


def _tc_fin(s_prev, p, cb, w2, b2, rblk):
    n_pad = s_prev.shape[0]

    def body(s_ref, p_ref, cb_ref, w_ref, b_ref, o_ref):
        xk = s_ref[...] + _relu(p_ref[0] + p_ref[1] + cb_ref[...])
        o_ref[...] = jnp.dot(xk, w_ref[...], precision=lax.Precision.HIGHEST,
                             preferred_element_type=jnp.float32) + b_ref[...]

    return pl.pallas_call(
        body,
        grid=(n_pad // rblk,),
        in_specs=[pl.BlockSpec((rblk, _F), lambda i: (i, 0)),
                  pl.BlockSpec((2, rblk, _F), lambda i: (0, i, 0)),
                  pl.BlockSpec((1, _F), lambda i: (0, 0)),
                  pl.BlockSpec((_F, 1), lambda i: (0, 0)),
                  pl.BlockSpec((1, 1), lambda i: (0, 0))],
        out_specs=pl.BlockSpec((rblk, 1), lambda i: (i, 0)),
        out_shape=jax.ShapeDtypeStruct((n_pad, 1), jnp.float32),
    )(s_prev, p, cb, w2, b2)


def kernel(x, edge_index,
           conv11_w, conv11_b, conv21_w, conv21_b, conv31_w, conv31_b,
           fc11_w, fc11_b, fc12_w, fc12_b, fc13_w, fc13_b,
           fc21_w, fc21_b, fc22_w, fc22_b, fc23_w, fc23_b,
           fc31_w, fc31_b, fc32_w, fc32_b, fc33_w, fc33_b,
           fc2_w, fc2_b):
    n, din = x.shape
    e = edge_index.shape[1]
    info = plsc.get_sparse_core_info()
    nc, ns = info.num_cores, info.num_subcores
    nw = nc * ns

    n_chunks = -(-e // (nw * _CHUNK))
    e_pad = nw * _CHUNK * n_chunks
    n_pad = -(-n // (8 * ns)) * (8 * ns)
    if e_pad > e and n_pad == n:
        n_pad += 8 * ns
    rblk = n_pad // 4

    src = edge_index[0]
    dst = edge_index[1]
    pad = e_pad - e
    if pad:
        src = jnp.concatenate([src, jnp.zeros((pad,), jnp.int32)])
        dst = jnp.concatenate([dst, jnp.full((pad,), n, jnp.int32)])
    src3 = src.reshape(nw, n_chunks, _CHUNK)
    dst3 = dst.reshape(nw, n_chunks, _CHUNK)

    xp = jnp.pad(x, ((0, n_pad - n), (0, 0)))
    zeros = jnp.zeros((n_pad, _F), jnp.float32)

    def wcat(a, b, c, d):
        return jnp.concatenate([a, b, c, d], axis=1)

    def bcat(a, b, c):
        return jnp.concatenate([a, b, c, jnp.zeros((_F,), jnp.float32)]).reshape(1, -1)

    wcat1 = wcat(fc11_w, fc12_w, fc13_w, conv11_w)
    bcat1 = bcat(fc11_b, fc12_b, fc13_b)
    wcat2 = wcat(fc21_w, fc22_w, fc23_w, conv21_w)
    bcat2 = bcat(fc21_b, fc22_b, fc23_b)
    wcat3 = wcat(fc31_w, fc32_w, fc33_w, conv31_w)
    bcat3 = bcat(fc31_b, fc32_b, fc33_b)

    y1, s1 = _tc_pre(xp, wcat1, bcat1, rblk)
    p1 = _seg_sum_sc(y1, src3, dst3, zeros, n_pad, n_chunks, nc, ns)
    y2, s2 = _tc_mid(s1, p1, conv11_b.reshape(1, -1), wcat2, bcat2, rblk)
    p2 = _seg_sum_sc(y2, src3, dst3, zeros, n_pad, n_chunks, nc, ns)
    y3, s3 = _tc_mid(s2, p2, conv21_b.reshape(1, -1), wcat3, bcat3, rblk)
    p3 = _seg_sum_sc(y3, src3, dst3, zeros, n_pad, n_chunks, nc, ns)
    out = _tc_fin(s3, p3, conv31_b.reshape(1, -1), fc2_w, fc2_b.reshape(1, 1), rblk)
    return out[:n]
